# baseline (speedup 1.0000x reference)
import dataclasses
import functools

import jax
import jax.numpy as jnp
from jax import lax
from jax.experimental import pallas as pl
from jax.experimental.pallas import tpu as pltpu
from jax.experimental.pallas import tpu_sc as plsc

N = 10000
E = 320000
NUM_GRAPHS = 64
IN_CH = 128
HID = 64

NC = 2
NS = 16
NW = NC * NS
CHUNK = 128

NPAD = 10112
E_PAD = 327680
N_CHUNKS = E_PAD // (NW * CHUNK)

DEG_W = 16
DPAD = 10240
CNT_OFF = NPAD
DEG_LIST = 344064
DEG_CHUNKS = DEG_LIST // (NW * CHUNK)


def _mesh():
    return plsc.VectorSubcoreMesh(core_axis_name="c", subcore_axis_name="s")


def _no_layout_cp():
    cp = pltpu.CompilerParams()
    if "needs_layout_passes" in pltpu.CompilerParams.__dataclass_fields__:
        cp = dataclasses.replace(cp, needs_layout_passes=False)
    return cp


HALF = DPAD // 2


def _count_kernel(idx3d, zdeg):
    @functools.partial(
        pl.kernel,
        out_type=jax.ShapeDtypeStruct((NW, 2, 16, HALF), jnp.float32),
        mesh=_mesh(),
        compiler_params=_no_layout_cp(),
        scratch_types=[
            pltpu.VMEM((DEG_CHUNKS, CHUNK), jnp.int32),
            pltpu.VMEM((16, HALF), jnp.float32),
        ],
    )
    def k(idx_hbm, z_hbm, out_hbm, didx, hist):
        c = lax.axis_index("c")
        s = lax.axis_index("s")
        wid = c * NS + s
        pltpu.sync_copy(idx_hbm.at[wid], didx)
        lane = lax.broadcasted_iota(jnp.int32, (16,), 0)
        ones16 = jnp.ones((16,), jnp.float32)

        @pl.loop(0, 2)
        def _(half):
            base = half * HALF
            pltpu.sync_copy(z_hbm, hist)

            @pl.loop(0, DEG_CHUNKS)
            def _(j):
                @pl.loop(0, CHUNK // 16)
                def _(g):
                    idx = didx[j, pl.ds(g * 16, 16)]
                    local = idx - base
                    m = (idx >= base) & (idx < base + HALF)
                    plsc.addupdate_scatter(hist, [lane, local], ones16,
                                           mask=m)

            pltpu.sync_copy(hist, out_hbm.at[wid].at[half])

    return k(idx3d, zdeg)


TOT_CHUNKS = E_PAD // CHUNK
SEG0_CHUNKS = 40
SEG0 = 4
C0_CHUNKS = SEG0 * SEG0_CHUNKS
assert NS * C0_CHUNKS == TOT_CHUNKS


def _edge_pass(hs, src2d, dst2d, zeros, F):
    NBUF = 2

    @functools.partial(
        pl.kernel,
        out_type=jax.ShapeDtypeStruct((NPAD, F), jnp.float32),
        mesh=_mesh(),
        scratch_types=(
            [pltpu.VMEM((SEG0_CHUNKS, CHUNK), jnp.int32),
             pltpu.VMEM((SEG0_CHUNKS, CHUNK), jnp.int32),
             pltpu.VMEM_SHARED((NPAD, F), jnp.float32)]
            + [pltpu.VMEM((CHUNK, F), jnp.float32)] * NBUF
            + [pltpu.SemaphoreType.DMA] * (2 * NBUF)
        ),
    )
    def k(hs_hbm, src_hbm, dst_hbm, z_hbm, out_hbm, sidx, didx, acc, *bufs):
        rows = bufs[:NBUF]
        gsem = bufs[NBUF:2 * NBUF]
        ssem = bufs[2 * NBUF:]
        c = lax.axis_index("c")
        s = lax.axis_index("s")
        rpt = NPAD // NS
        base = s * rpt

        def run_segment(off, seg_chunks):
            pltpu.sync_copy(src_hbm.at[pl.ds(off, seg_chunks)], sidx)
            pltpu.sync_copy(dst_hbm.at[pl.ds(off, seg_chunks)], didx)
            for b in range(NBUF):
                pltpu.async_copy(hs_hbm.at[sidx.at[b]], rows[b], gsem[b])

            @pl.loop(0, seg_chunks, step=NBUF)
            def _(j):
                scs = []
                for b in range(NBUF):
                    pltpu.make_async_copy(hs_hbm.at[sidx.at[j + b]],
                                          rows[b], gsem[b]).wait()
                    scs.append(
                        pltpu.async_copy(rows[b], acc.at[didx.at[j + b]],
                                         ssem[b], add=True))
                for b in range(NBUF):
                    scs[b].wait()

                    @pl.when(j + b + NBUF < seg_chunks)
                    def _(b=b):
                        pltpu.async_copy(hs_hbm.at[sidx.at[j + b + NBUF]],
                                         rows[b], gsem[b])

        @pl.when(c == 0)
        def _():
            pltpu.sync_copy(z_hbm.at[pl.ds(base, rpt)],
                            acc.at[pl.ds(base, rpt)])
            plsc.subcore_barrier()
            for seg in range(SEG0):
                run_segment(s * C0_CHUNKS + seg * SEG0_CHUNKS, SEG0_CHUNKS)
            plsc.subcore_barrier()
            pltpu.sync_copy(acc.at[pl.ds(base, rpt)],
                            out_hbm.at[pl.ds(base, rpt)])

    return k(hs, src2d, dst2d, zeros)


def _tc(fn, out_shape, *args):
    return pl.pallas_call(fn, out_shape=out_shape)(*args)


def _mm1_body(x_ref, w_ref, o_ref):
    o_ref[...] = jnp.dot(x_ref[...], w_ref[...],
                         preferred_element_type=jnp.float32)


def _scale_body(h_ref, deg_ref, hs_ref, dis_ref, cnt_ref):
    d = jnp.sum(deg_ref[...], axis=0)
    ones161 = jnp.ones((16, 1), jnp.float32)
    col0 = lax.dot_general(d[0], ones161, (((0,), (0,)), ((), ())),
                           preferred_element_type=jnp.float32)
    col1 = lax.dot_general(d[1], ones161, (((0,), (0,)), ((), ())),
                           preferred_element_type=jnp.float32)
    deg = jnp.concatenate([col0, col1], axis=0)
    dis = lax.rsqrt(1.0 + deg[:N])
    dis_ref[...] = dis
    cnt_ref[...] = deg[CNT_OFF:CNT_OFF + NUM_GRAPHS]
    hs_ref[...] = jnp.concatenate(
        [h_ref[...] * dis, jnp.zeros((N, 2 * HID - HID), jnp.float32)], axis=1)


def _layer_body(nf, s_ref, hs_ref, dis_ref, b_ref, w_ref, o_ref):
    sarr = s_ref[...]
    agg = sarr[:N, :nf] + hs_ref[:, :nf]
    dis = dis_ref[...]
    a = jnp.maximum(dis * agg + b_ref[...], 0.0)
    o_ref[...] = jnp.dot(a, w_ref[...],
                         preferred_element_type=jnp.float32) * dis


def _final_body(s_ref, hs_ref, dis_ref, b_ref, cnt_ref, batch_ref,
                wf1_ref, bf1_ref, wf2_ref, bf2_ref, o_ref):
    sarr = s_ref[...]
    out3 = dis_ref[...] * (sarr[:N, :] + hs_ref[...]) + b_ref[...]
    gids = lax.broadcasted_iota(jnp.int32, (1, NUM_GRAPHS), 1)
    onehot = (batch_ref[...] == gids).astype(jnp.float32)
    sums = lax.dot_general(onehot, out3, (((0,), (0,)), ((), ())),
                           preferred_element_type=jnp.float32)
    pooled = sums / jnp.maximum(cnt_ref[...], 1.0)
    z = jnp.maximum(jnp.dot(pooled, wf1_ref[...],
                            preferred_element_type=jnp.float32)
                    + bf1_ref[...], 0.0)
    o_ref[...] = jnp.dot(z, wf2_ref[...],
                         preferred_element_type=jnp.float32) + bf2_ref[...]


@jax.jit
def kernel(x, edge_index, batch, W1, b1, W2, b2, W3, b3, Wf1, bf1, Wf2, bf2):
    src = edge_index[0]
    dst = edge_index[1]

    pad_e = E_PAD - E
    src_p = jnp.concatenate([src, jnp.zeros((pad_e,), jnp.int32)])
    dst_p = jnp.concatenate([dst, jnp.full((pad_e,), N, jnp.int32)])
    src2d = src_p.reshape(TOT_CHUNKS, CHUNK)
    dst2d = dst_p.reshape(TOT_CHUNKS, CHUNK)

    pad_c = DEG_LIST - E - N
    cnt_idx = jnp.concatenate([
        dst, batch + CNT_OFF, jnp.full((pad_c,), N, jnp.int32)
    ]).reshape(NW, DEG_CHUNKS, CHUNK)

    zdeg = jnp.zeros((16, HALF), jnp.float32)
    z128 = jnp.zeros((NPAD, 2 * HID), jnp.float32)
    batch2d = batch.reshape(N, 1)

    counts = _count_kernel(cnt_idx, zdeg)
    h1 = _tc(_mm1_body, jax.ShapeDtypeStruct((N, HID), jnp.float32), x, W1)

    hs1, dis, cnt = _tc(
        _scale_body,
        (jax.ShapeDtypeStruct((N, 2 * HID), jnp.float32),
         jax.ShapeDtypeStruct((N, 1), jnp.float32),
         jax.ShapeDtypeStruct((NUM_GRAPHS, 1), jnp.float32)),
        h1, counts)

    s1 = _edge_pass(hs1, src2d, dst2d, z128, 2 * HID)
    hs2 = _tc(functools.partial(_layer_body, HID),
              jax.ShapeDtypeStruct((N, 2 * HID), jnp.float32),
              s1, hs1, dis, b1.reshape(1, HID), W2)

    s2 = _edge_pass(hs2, src2d, dst2d, z128, 2 * HID)
    hs3 = _tc(functools.partial(_layer_body, 2 * HID),
              jax.ShapeDtypeStruct((N, 2 * HID), jnp.float32),
              s2, hs2, dis, b2.reshape(1, 2 * HID), W3)

    s3 = _edge_pass(hs3, src2d, dst2d, z128, 2 * HID)
    out = _tc(_final_body, jax.ShapeDtypeStruct((NUM_GRAPHS, 1), jnp.float32),
              s3, hs3, dis, b3.reshape(1, 2 * HID), cnt, batch2d,
              Wf1, bf1.reshape(1, HID), Wf2, bf2.reshape(1, 1))
    return out


# speedup vs baseline: 1.5408x; 1.5408x over previous
import dataclasses
import functools

import jax
import jax.numpy as jnp
from jax import lax
from jax.experimental import pallas as pl
from jax.experimental.pallas import tpu as pltpu
from jax.experimental.pallas import tpu_sc as plsc

N = 10000
E = 320000
NUM_GRAPHS = 64
IN_CH = 128
HID = 64

NC = 2
NS = 16
NW = NC * NS
CHUNK = 128

NPAD = 10112
E_PAD = 327680
N_CHUNKS = E_PAD // (NW * CHUNK)

DEG_W = 16
DPAD = 10240
CNT_OFF = NPAD
DEG_LIST = 344064
DEG_CHUNKS = DEG_LIST // (NW * CHUNK)


def _mesh():
    return plsc.VectorSubcoreMesh(core_axis_name="c", subcore_axis_name="s")


def _no_layout_cp():
    cp = pltpu.CompilerParams()
    if "needs_layout_passes" in pltpu.CompilerParams.__dataclass_fields__:
        cp = dataclasses.replace(cp, needs_layout_passes=False)
    return cp


HALF = DPAD // 2


def _count_kernel(idx3d, zdeg):
    @functools.partial(
        pl.kernel,
        out_type=jax.ShapeDtypeStruct((NW, 2, 16, HALF), jnp.float32),
        mesh=_mesh(),
        compiler_params=_no_layout_cp(),
        scratch_types=[
            pltpu.VMEM((DEG_CHUNKS, CHUNK), jnp.int32),
            pltpu.VMEM((16, HALF), jnp.float32),
        ],
    )
    def k(idx_hbm, z_hbm, out_hbm, didx, hist):
        c = lax.axis_index("c")
        s = lax.axis_index("s")
        wid = c * NS + s
        pltpu.sync_copy(idx_hbm.at[wid], didx)
        lane = lax.broadcasted_iota(jnp.int32, (16,), 0)
        ones16 = jnp.ones((16,), jnp.float32)

        @pl.loop(0, 2)
        def _(half):
            base = half * HALF
            pltpu.sync_copy(z_hbm, hist)

            @pl.loop(0, DEG_CHUNKS)
            def _(j):
                @pl.loop(0, CHUNK // 16)
                def _(g):
                    idx = didx[j, pl.ds(g * 16, 16)]
                    local = idx - base
                    m = (idx >= base) & (idx < base + HALF)
                    plsc.addupdate_scatter(hist, [lane, local], ones16,
                                           mask=m)

            pltpu.sync_copy(hist, out_hbm.at[wid].at[half])

    return k(idx3d, zdeg)


TOT_CHUNKS = E_PAD // CHUNK
SEG0_LIST = (48, 48, 48, 8)
C0_CHUNKS = sum(SEG0_LIST)
SEG1_LIST = (8,)
C1_CHUNKS = sum(SEG1_LIST)
MAXSEG = max(max(SEG0_LIST), max(SEG1_LIST))
assert NS * (C0_CHUNKS + C1_CHUNKS) == TOT_CHUNKS


def _edge_pass(hs, src2d, dst2d, zeros, F):
    NBUF = 2

    @functools.partial(
        pl.kernel,
        out_type=jax.ShapeDtypeStruct((NC, NPAD, F), jnp.float32),
        mesh=_mesh(),
        scratch_types=(
            [pltpu.VMEM((MAXSEG, CHUNK), jnp.int32),
             pltpu.VMEM((MAXSEG, CHUNK), jnp.int32),
             pltpu.VMEM_SHARED((NPAD, F), jnp.float32)]
            + [pltpu.VMEM((CHUNK, F), jnp.float32)] * NBUF
            + [pltpu.SemaphoreType.DMA] * (2 * NBUF)
        ),
    )
    def k(hs_hbm, src_hbm, dst_hbm, z_hbm, out_hbm, sidx, didx, acc, *bufs):
        rows = bufs[:NBUF]
        gsem = bufs[NBUF:2 * NBUF]
        ssem = bufs[2 * NBUF:]
        c = lax.axis_index("c")
        s = lax.axis_index("s")
        rpt = NPAD // NS
        base = s * rpt

        def run_segment(off, seg_chunks):
            pltpu.sync_copy(src_hbm.at[pl.ds(off, seg_chunks)],
                            sidx.at[pl.ds(0, seg_chunks)])
            pltpu.sync_copy(dst_hbm.at[pl.ds(off, seg_chunks)],
                            didx.at[pl.ds(0, seg_chunks)])
            for b in range(NBUF):
                pltpu.async_copy(hs_hbm.at[sidx.at[b]], rows[b], gsem[b])

            @pl.loop(0, seg_chunks, step=NBUF)
            def _(j):
                scs = []
                for b in range(NBUF):
                    pltpu.make_async_copy(hs_hbm.at[sidx.at[j + b]],
                                          rows[b], gsem[b]).wait()
                    scs.append(
                        pltpu.async_copy(rows[b], acc.at[didx.at[j + b]],
                                         ssem[b], add=True))
                for b in range(NBUF):
                    scs[b].wait()

                    @pl.when(j + b + NBUF < seg_chunks)
                    def _(b=b):
                        pltpu.async_copy(hs_hbm.at[sidx.at[j + b + NBUF]],
                                         rows[b], gsem[b])

        pltpu.sync_copy(z_hbm.at[pl.ds(base, rpt)], acc.at[pl.ds(base, rpt)])
        plsc.subcore_barrier()

        @pl.when(c == 0)
        def _():
            off = s * C0_CHUNKS
            for seg in SEG0_LIST:
                run_segment(off, seg)
                off += seg

        @pl.when(c == 1)
        def _():
            off = NS * C0_CHUNKS + s * C1_CHUNKS
            for seg in SEG1_LIST:
                run_segment(off, seg)
                off += seg

        plsc.subcore_barrier()
        pltpu.sync_copy(acc.at[pl.ds(base, rpt)],
                        out_hbm.at[c].at[pl.ds(base, rpt)])

    return k(hs, src2d, dst2d, zeros)


def _tc(fn, out_shape, *args):
    return pl.pallas_call(fn, out_shape=out_shape)(*args)


def _mm1_body(x_ref, w_ref, o_ref):
    o_ref[...] = jnp.dot(x_ref[...], w_ref[...],
                         preferred_element_type=jnp.float32)


def _scale_body(h_ref, deg_ref, hs_ref, dis_ref, cnt_ref):
    d = jnp.sum(deg_ref[...], axis=0)
    ones161 = jnp.ones((16, 1), jnp.float32)
    col0 = lax.dot_general(d[0], ones161, (((0,), (0,)), ((), ())),
                           preferred_element_type=jnp.float32)
    col1 = lax.dot_general(d[1], ones161, (((0,), (0,)), ((), ())),
                           preferred_element_type=jnp.float32)
    deg = jnp.concatenate([col0, col1], axis=0)
    dis = lax.rsqrt(1.0 + deg[:N])
    dis_ref[...] = dis
    cnt_ref[...] = deg[CNT_OFF:CNT_OFF + NUM_GRAPHS]
    hs_ref[...] = jnp.concatenate(
        [h_ref[...] * dis, jnp.zeros((N, 2 * HID - HID), jnp.float32)], axis=1)


def _layer_body(nf, s_ref, hs_ref, dis_ref, b_ref, w_ref, o_ref):
    sarr = s_ref[...]
    agg = sarr[0, :N, :nf] + sarr[1, :N, :nf] + hs_ref[:, :nf]
    dis = dis_ref[...]
    a = jnp.maximum(dis * agg + b_ref[...], 0.0)
    o_ref[...] = jnp.dot(a, w_ref[...],
                         preferred_element_type=jnp.float32) * dis


def _final_body(s_ref, hs_ref, dis_ref, b_ref, cnt_ref, batch_ref,
                wf1_ref, bf1_ref, wf2_ref, bf2_ref, o_ref):
    sarr = s_ref[...]
    out3 = dis_ref[...] * (sarr[0, :N, :] + sarr[1, :N, :] + hs_ref[...]) \
        + b_ref[...]
    gids = lax.broadcasted_iota(jnp.int32, (1, NUM_GRAPHS), 1)
    onehot = (batch_ref[...] == gids).astype(jnp.float32)
    sums = lax.dot_general(onehot, out3, (((0,), (0,)), ((), ())),
                           preferred_element_type=jnp.float32)
    pooled = sums / jnp.maximum(cnt_ref[...], 1.0)
    z = jnp.maximum(jnp.dot(pooled, wf1_ref[...],
                            preferred_element_type=jnp.float32)
                    + bf1_ref[...], 0.0)
    o_ref[...] = jnp.dot(z, wf2_ref[...],
                         preferred_element_type=jnp.float32) + bf2_ref[...]


@jax.jit
def kernel(x, edge_index, batch, W1, b1, W2, b2, W3, b3, Wf1, bf1, Wf2, bf2):
    src = edge_index[0]
    dst = edge_index[1]

    pad_e = E_PAD - E
    src_p = jnp.concatenate([src, jnp.zeros((pad_e,), jnp.int32)])
    dst_p = jnp.concatenate([dst, jnp.full((pad_e,), N, jnp.int32)])
    src2d = src_p.reshape(TOT_CHUNKS, CHUNK)
    dst2d = dst_p.reshape(TOT_CHUNKS, CHUNK)

    pad_c = DEG_LIST - E - N
    cnt_idx = jnp.concatenate([
        dst, batch + CNT_OFF, jnp.full((pad_c,), N, jnp.int32)
    ]).reshape(NW, DEG_CHUNKS, CHUNK)

    zdeg = jnp.zeros((16, HALF), jnp.float32)
    z128 = jnp.zeros((NPAD, 2 * HID), jnp.float32)
    batch2d = batch.reshape(N, 1)

    counts = _count_kernel(cnt_idx, zdeg)
    h1 = _tc(_mm1_body, jax.ShapeDtypeStruct((N, HID), jnp.float32), x, W1)

    hs1, dis, cnt = _tc(
        _scale_body,
        (jax.ShapeDtypeStruct((N, 2 * HID), jnp.float32),
         jax.ShapeDtypeStruct((N, 1), jnp.float32),
         jax.ShapeDtypeStruct((NUM_GRAPHS, 1), jnp.float32)),
        h1, counts)

    s1 = _edge_pass(hs1, src2d, dst2d, z128, 2 * HID)
    hs2 = _tc(functools.partial(_layer_body, HID),
              jax.ShapeDtypeStruct((N, 2 * HID), jnp.float32),
              s1, hs1, dis, b1.reshape(1, HID), W2)

    s2 = _edge_pass(hs2, src2d, dst2d, z128, 2 * HID)
    hs3 = _tc(functools.partial(_layer_body, 2 * HID),
              jax.ShapeDtypeStruct((N, 2 * HID), jnp.float32),
              s2, hs2, dis, b2.reshape(1, 2 * HID), W3)

    s3 = _edge_pass(hs3, src2d, dst2d, z128, 2 * HID)
    out = _tc(_final_body, jax.ShapeDtypeStruct((NUM_GRAPHS, 1), jnp.float32),
              s3, hs3, dis, b3.reshape(1, 2 * HID), cnt, batch2d,
              Wf1, bf1.reshape(1, HID), Wf2, bf2.reshape(1, 1))
    return out


# speedup vs baseline: 1.5543x; 1.0088x over previous
import dataclasses
import functools

import jax
import jax.numpy as jnp
from jax import lax
from jax.experimental import pallas as pl
from jax.experimental.pallas import tpu as pltpu
from jax.experimental.pallas import tpu_sc as plsc

N = 10000
E = 320000
NUM_GRAPHS = 64
IN_CH = 128
HID = 64

NC = 2
NS = 16
NW = NC * NS
CHUNK = 128

NPAD = 10112
E_PAD = 327680
N_CHUNKS = E_PAD // (NW * CHUNK)

DEG_W = 16
DPAD = 10240
CNT_OFF = NPAD
DEG_LIST = 344064
DEG_CHUNKS = DEG_LIST // (NW * CHUNK)


def _mesh():
    return plsc.VectorSubcoreMesh(core_axis_name="c", subcore_axis_name="s")


def _no_layout_cp():
    cp = pltpu.CompilerParams()
    if "needs_layout_passes" in pltpu.CompilerParams.__dataclass_fields__:
        cp = dataclasses.replace(cp, needs_layout_passes=False)
    return cp


HALF = DPAD // 2


def _count_kernel(idx3d, zdeg):
    @functools.partial(
        pl.kernel,
        out_type=jax.ShapeDtypeStruct((NW, 2, 16, HALF), jnp.float32),
        mesh=_mesh(),
        compiler_params=_no_layout_cp(),
        scratch_types=[
            pltpu.VMEM((DEG_CHUNKS, CHUNK), jnp.int32),
            pltpu.VMEM((16, HALF), jnp.float32),
        ],
    )
    def k(idx_hbm, z_hbm, out_hbm, didx, hist):
        c = lax.axis_index("c")
        s = lax.axis_index("s")
        wid = c * NS + s
        pltpu.sync_copy(idx_hbm.at[wid], didx)
        lane = lax.broadcasted_iota(jnp.int32, (16,), 0)
        ones16 = jnp.ones((16,), jnp.float32)

        @pl.loop(0, 2)
        def _(half):
            base = half * HALF
            pltpu.sync_copy(z_hbm, hist)

            @pl.loop(0, DEG_CHUNKS)
            def _(j):
                @pl.loop(0, CHUNK // 16)
                def _(g):
                    idx = didx[j, pl.ds(g * 16, 16)]
                    local = idx - base
                    m = (idx >= base) & (idx < base + HALF)
                    plsc.addupdate_scatter(hist, [lane, local], ones16,
                                           mask=m)

            pltpu.sync_copy(hist, out_hbm.at[wid].at[half])

    return k(idx3d, zdeg)


TOT_CHUNKS = E_PAD // CHUNK
SEG0_LIST = (48, 48, 48, 8)
C0_CHUNKS = sum(SEG0_LIST)
SEG1_LIST = (8,)
C1_CHUNKS = sum(SEG1_LIST)
MAXSEG = max(max(SEG0_LIST), max(SEG1_LIST))
assert NS * (C0_CHUNKS + C1_CHUNKS) == TOT_CHUNKS


def _edge_pass(hs, src2d, dst2d, F):
    NBUF = 2

    @functools.partial(
        pl.kernel,
        out_type=jax.ShapeDtypeStruct((NC, NPAD, F), jnp.float32),
        mesh=_mesh(),
        scratch_types=(
            [pltpu.VMEM((MAXSEG, CHUNK), jnp.int32),
             pltpu.VMEM((MAXSEG, CHUNK), jnp.int32),
             pltpu.VMEM_SHARED((NPAD, F), jnp.float32)]
            + [pltpu.VMEM((CHUNK, F), jnp.float32)] * NBUF
            + [pltpu.SemaphoreType.DMA] * (2 * NBUF)
        ),
    )
    def k(hs_hbm, src_hbm, dst_hbm, out_hbm, sidx, didx, acc, *bufs):
        rows = bufs[:NBUF]
        gsem = bufs[NBUF:2 * NBUF]
        ssem = bufs[2 * NBUF:]
        c = lax.axis_index("c")
        s = lax.axis_index("s")
        rpt = NPAD // NS
        base = s * rpt

        def run_segment(off, seg_chunks):
            pltpu.sync_copy(src_hbm.at[pl.ds(off, seg_chunks)],
                            sidx.at[pl.ds(0, seg_chunks)])
            pltpu.sync_copy(dst_hbm.at[pl.ds(off, seg_chunks)],
                            didx.at[pl.ds(0, seg_chunks)])
            for b in range(NBUF):
                pltpu.async_copy(hs_hbm.at[sidx.at[b]], rows[b], gsem[b])

            @pl.loop(0, seg_chunks, step=NBUF)
            def _(j):
                scs = []
                for b in range(NBUF):
                    pltpu.make_async_copy(hs_hbm.at[sidx.at[j + b]],
                                          rows[b], gsem[b]).wait()
                    scs.append(
                        pltpu.async_copy(rows[b], acc.at[didx.at[j + b]],
                                         ssem[b], add=True))
                for b in range(NBUF):
                    scs[b].wait()

                    @pl.when(j + b + NBUF < seg_chunks)
                    def _(b=b):
                        pltpu.async_copy(hs_hbm.at[sidx.at[j + b + NBUF]],
                                         rows[b], gsem[b])

        z16 = jnp.zeros((16,), jnp.float32)
        zrow = rows[0]

        @pl.loop(0, CHUNK)
        def _(i):
            @pl.loop(0, F // 16)
            def _(g):
                zrow[i, pl.ds(g * 16, 16)] = z16

        nfull = rpt // CHUNK
        rem = rpt - nfull * CHUNK
        for kk in range(nfull):
            pltpu.sync_copy(zrow, acc.at[pl.ds(base + kk * CHUNK, CHUNK)])
        if rem:
            pltpu.sync_copy(zrow.at[pl.ds(0, rem)],
                            acc.at[pl.ds(base + nfull * CHUNK, rem)])
        plsc.subcore_barrier()

        @pl.when(c == 0)
        def _():
            off = s * C0_CHUNKS
            for seg in SEG0_LIST:
                run_segment(off, seg)
                off += seg

        @pl.when(c == 1)
        def _():
            off = NS * C0_CHUNKS + s * C1_CHUNKS
            for seg in SEG1_LIST:
                run_segment(off, seg)
                off += seg

        plsc.subcore_barrier()
        pltpu.sync_copy(acc.at[pl.ds(base, rpt)],
                        out_hbm.at[c].at[pl.ds(base, rpt)])

    return k(hs, src2d, dst2d)


def _tc(fn, out_shape, *args):
    return pl.pallas_call(fn, out_shape=out_shape)(*args)


def _mm1_body(x_ref, w_ref, o_ref):
    o_ref[...] = jnp.dot(x_ref[...], w_ref[...],
                         preferred_element_type=jnp.float32)


def _scale_body(h_ref, deg_ref, hs_ref, dis_ref, cnt_ref):
    d = jnp.sum(deg_ref[...], axis=0)
    ones161 = jnp.ones((16, 1), jnp.float32)
    col0 = lax.dot_general(d[0], ones161, (((0,), (0,)), ((), ())),
                           preferred_element_type=jnp.float32)
    col1 = lax.dot_general(d[1], ones161, (((0,), (0,)), ((), ())),
                           preferred_element_type=jnp.float32)
    deg = jnp.concatenate([col0, col1], axis=0)
    dis = lax.rsqrt(1.0 + deg[:N])
    dis_ref[...] = dis
    cnt_ref[...] = deg[CNT_OFF:CNT_OFF + NUM_GRAPHS]
    hs_ref[...] = jnp.concatenate(
        [h_ref[...] * dis, jnp.zeros((N, 2 * HID - HID), jnp.float32)], axis=1)


def _layer_body(nf, s_ref, hs_ref, dis_ref, b_ref, w_ref, o_ref):
    sarr = s_ref[...]
    agg = sarr[0, :N, :nf] + sarr[1, :N, :nf] + hs_ref[:, :nf]
    dis = dis_ref[...]
    a = jnp.maximum(dis * agg + b_ref[...], 0.0)
    o_ref[...] = jnp.dot(a, w_ref[...],
                         preferred_element_type=jnp.float32) * dis


def _final_body(s_ref, hs_ref, dis_ref, b_ref, cnt_ref, batch_ref,
                wf1_ref, bf1_ref, wf2_ref, bf2_ref, o_ref):
    sarr = s_ref[...]
    out3 = dis_ref[...] * (sarr[0, :N, :] + sarr[1, :N, :] + hs_ref[...]) \
        + b_ref[...]
    gids = lax.broadcasted_iota(jnp.int32, (1, NUM_GRAPHS), 1)
    onehot = (batch_ref[...] == gids).astype(jnp.float32)
    sums = lax.dot_general(onehot, out3, (((0,), (0,)), ((), ())),
                           preferred_element_type=jnp.float32)
    pooled = sums / jnp.maximum(cnt_ref[...], 1.0)
    z = jnp.maximum(jnp.dot(pooled, wf1_ref[...],
                            preferred_element_type=jnp.float32)
                    + bf1_ref[...], 0.0)
    o_ref[...] = jnp.dot(z, wf2_ref[...],
                         preferred_element_type=jnp.float32) + bf2_ref[...]


@jax.jit
def kernel(x, edge_index, batch, W1, b1, W2, b2, W3, b3, Wf1, bf1, Wf2, bf2):
    src = edge_index[0]
    dst = edge_index[1]

    pad_e = E_PAD - E
    src_p = jnp.concatenate([src, jnp.zeros((pad_e,), jnp.int32)])
    dst_p = jnp.concatenate([dst, jnp.full((pad_e,), N, jnp.int32)])
    src2d = src_p.reshape(TOT_CHUNKS, CHUNK)
    dst2d = dst_p.reshape(TOT_CHUNKS, CHUNK)

    pad_c = DEG_LIST - E - N
    cnt_idx = jnp.concatenate([
        dst, batch + CNT_OFF, jnp.full((pad_c,), N, jnp.int32)
    ]).reshape(NW, DEG_CHUNKS, CHUNK)

    zdeg = jnp.zeros((16, HALF), jnp.float32)
    batch2d = batch.reshape(N, 1)

    counts = _count_kernel(cnt_idx, zdeg)
    h1 = _tc(_mm1_body, jax.ShapeDtypeStruct((N, HID), jnp.float32), x, W1)

    hs1, dis, cnt = _tc(
        _scale_body,
        (jax.ShapeDtypeStruct((N, 2 * HID), jnp.float32),
         jax.ShapeDtypeStruct((N, 1), jnp.float32),
         jax.ShapeDtypeStruct((NUM_GRAPHS, 1), jnp.float32)),
        h1, counts)

    s1 = _edge_pass(hs1, src2d, dst2d, 2 * HID)
    hs2 = _tc(functools.partial(_layer_body, HID),
              jax.ShapeDtypeStruct((N, 2 * HID), jnp.float32),
              s1, hs1, dis, b1.reshape(1, HID), W2)

    s2 = _edge_pass(hs2, src2d, dst2d, 2 * HID)
    hs3 = _tc(functools.partial(_layer_body, 2 * HID),
              jax.ShapeDtypeStruct((N, 2 * HID), jnp.float32),
              s2, hs2, dis, b2.reshape(1, 2 * HID), W3)

    s3 = _edge_pass(hs3, src2d, dst2d, 2 * HID)
    out = _tc(_final_body, jax.ShapeDtypeStruct((NUM_GRAPHS, 1), jnp.float32),
              s3, hs3, dis, b3.reshape(1, 2 * HID), cnt, batch2d,
              Wf1, bf1.reshape(1, HID), Wf2, bf2.reshape(1, 1))
    return out
